# baseline (speedup 1.0000x reference)
import jax
import jax.numpy as jnp
from jax import lax
from jax.experimental import pallas as pl
from jax.experimental.pallas import tpu as pltpu
from jax.experimental.pallas import tpu_sc as plsc

N = 10000
E = 320000
D = 128
NC = 2
NS = 16
NP = 10240
NB = NP
L = 16
K = 80

BLK = 2048
GRID = NP // BLK


def _mesh():
    return plsc.VectorSubcoreMesh(
        core_axis_name="c", subcore_axis_name="s", num_cores=NC, num_subcores=NS
    )


GB = 5
NG = (E // (NC * NS)) // K // GB


def _deg_body(src_hbm, dst_hbm, out_hbm, ibS, ibD, obS, obD, zb, isem, ssem, hist):
    c = lax.axis_index("c")
    s = lax.axis_index("s")
    oh0 = jnp.where(lax.iota(jnp.int32, L) == 0, 1.0, 0.0).astype(jnp.float32)
    oh1 = jnp.where(lax.iota(jnp.int32, L) == 1, 1.0, 0.0).astype(jnp.float32)
    zv = jnp.zeros((L,), jnp.float32)

    def fill_o(i, _):
        for j in range(D // L):
            obS[i, pl.ds(j * L, L)] = oh0 if j == 0 else zv
            obD[i, pl.ds(j * L, L)] = oh1 if j == 0 else zv
        return 0

    lax.fori_loop(0, K, fill_o, 0)

    def fill_z(i, _):
        for j in range(D // L):
            zb[i, pl.ds(j * L, L)] = zv
        return 0

    lax.fori_loop(0, 64, fill_z, 0)

    def fill_deg(i, _):
        pltpu.sync_copy(zb, hist.at[pl.ds(s * 640 + i * 64, 64), :])
        return 0

    lax.fori_loop(0, 10, fill_deg, 0)
    plsc.subcore_barrier()

    ept = E // (NC * NS)
    base_t = c * (E // NC) + s * ept

    def group(g, _):
        par = lax.rem(g, 2) * GB
        descs = []
        for b in range(GB):
            base = base_t + (g * GB + b) * K
            descs.append(pltpu.async_copy(src_hbm.at[pl.ds(base, K)], ibS.at[par + b], isem))
            descs.append(pltpu.async_copy(dst_hbm.at[pl.ds(base, K)], ibD.at[par + b], isem))

        for d_ in descs:
            d_.wait()
        for b in range(GB):
            pltpu.async_copy(obS, hist.at[ibS.at[par + b]], ssem, add=True)
            pltpu.async_copy(obD, hist.at[ibD.at[par + b]], ssem, add=True)

        @pl.when(g > 0)
        def _():
            for b in range(2 * GB):
                pltpu.make_async_copy(out_hbm.at[c, pl.ds(0, K), :], obS, ssem).wait()
        return 0

    lax.fori_loop(0, NG, group, 0)
    for b in range(2 * GB):
        pltpu.make_async_copy(out_hbm.at[c, pl.ds(0, K), :], obS, ssem).wait()
    plsc.subcore_barrier()
    sl = pl.ds(s * 640, 640)
    pltpu.sync_copy(hist.at[sl, :], out_hbm.at[c, sl, :])


def _deg_pass(src, dst):
    f = pl.kernel(
        _deg_body,
        out_type=jax.ShapeDtypeStruct((NC, NB, D), jnp.float32),
        mesh=_mesh(),
        scratch_types=[
            pltpu.VMEM((2 * GB, K), jnp.int32),
            pltpu.VMEM((2 * GB, K), jnp.int32),
            pltpu.VMEM((K, D), jnp.float32),
            pltpu.VMEM((K, D), jnp.float32),
            pltpu.VMEM((64, D), jnp.float32),
            pltpu.SemaphoreType.DMA,
            pltpu.SemaphoreType.DMA,
            pltpu.VMEM_SHARED((NB, D), jnp.float32),
        ],
    )
    return f(src, dst)


GBE = 4
NGE = 31


def _edge_body(h_hbm, src_hbm, dst_hbm, agg_hbm, sidx, didx, r0, r1, r2, r3,
               isem, gsem, ssem, asp):
    c = lax.axis_index("c")
    s = lax.axis_index("s")
    rows = [r0, r1, r2, r3]
    rpt = NP // NS
    zv = jnp.zeros((L,), jnp.float32)

    def fill_z(i, _):
        for j in range(D // L):
            r0[i, pl.ds(j * L, L)] = zv
        return 0

    lax.fori_loop(0, K, fill_z, 0)

    def fill_a(i, _):
        pltpu.sync_copy(r0, asp.at[pl.ds(s * rpt + i * K, K), :])
        return 0

    lax.fori_loop(0, 8, fill_a, 0)
    plsc.subcore_barrier()

    ept = E // (NC * NS)
    base_t = c * (E // NC) + s * ept

    def group(g, _):
        par = lax.rem(g, 2) * GBE
        descs = []
        for b in range(GBE):
            base = base_t + (g * GBE + b) * K
            descs.append(pltpu.async_copy(src_hbm.at[pl.ds(base, K)], sidx.at[par + b], isem))
            descs.append(pltpu.async_copy(dst_hbm.at[pl.ds(base, K)], didx.at[par + b], isem))
        for d_ in descs:
            d_.wait()
        gds = []
        for b in range(GBE):
            @pl.when(g > 0)
            def _():
                pltpu.make_async_copy(h_hbm.at[pl.ds(0, K), :], rows[b], ssem).wait()
            gds.append(pltpu.async_copy(h_hbm.at[sidx.at[par + b]], rows[b], gsem))
        for b in range(GBE):
            gds[b].wait()
            pltpu.async_copy(rows[b], asp.at[didx.at[par + b]], ssem, add=True)
        return 0

    lax.fori_loop(0, NGE, group, 0)
    for b in range(GBE):
        pltpu.make_async_copy(h_hbm.at[pl.ds(0, K), :], rows[b], ssem).wait()
    tbase = base_t + NGE * GBE * K
    pltpu.sync_copy(src_hbm.at[pl.ds(tbase, K)], sidx.at[0])
    pltpu.sync_copy(dst_hbm.at[pl.ds(tbase, K)], didx.at[0])
    pltpu.async_copy(h_hbm.at[sidx.at[0]], r0, gsem).wait()
    pltpu.async_copy(r0, asp.at[didx.at[0]], ssem, add=True).wait()
    plsc.subcore_barrier()
    sl = pl.ds(s * rpt, rpt)
    pltpu.sync_copy(asp.at[sl, :], agg_hbm.at[c, sl, :])


def _edge_pass(h, src, dst):
    f = pl.kernel(
        _edge_body,
        out_type=jax.ShapeDtypeStruct((NC, NP, D), jnp.float32),
        mesh=_mesh(),
        scratch_types=[
            pltpu.VMEM((2 * GBE, K), jnp.int32),
            pltpu.VMEM((2 * GBE, K), jnp.int32),
            pltpu.VMEM((K, D), jnp.float32),
            pltpu.VMEM((K, D), jnp.float32),
            pltpu.VMEM((K, D), jnp.float32),
            pltpu.VMEM((K, D), jnp.float32),
            pltpu.SemaphoreType.DMA,
            pltpu.SemaphoreType.DMA,
            pltpu.SemaphoreType.DMA,
            pltpu.VMEM_SHARED((NP, D), jnp.float32),
        ],
    )
    return f(h, src, dst)


def _norms(degp_ref, dsl):
    blk = degp_ref[0, dsl, :] + degp_ref[1, dsl, :]
    lanes = lax.broadcasted_iota(jnp.int32, (1, D), 1)
    d_out = jnp.sum(jnp.where(lanes == 0, blk, 0.0), axis=1)
    d_in = jnp.sum(jnp.where(lanes == 1, blk, 0.0), axis=1)
    no = jnp.where(d_out > 0, lax.rsqrt(jnp.maximum(d_out, 1.0)), 0.0)
    ni = jnp.where(d_in > 0, lax.rsqrt(jnp.maximum(d_in, 1.0)), 0.0)
    return no, ni


def _pre_body(x_ref, w_ref, degp_ref, h_ref):
    i = pl.program_id(0)
    dsl = pl.ds(i * BLK, BLK)
    no, _ = _norms(degp_ref, dsl)
    y = jnp.dot(x_ref[...], w_ref[...], preferred_element_type=jnp.float32)
    h_ref[...] = y * no[:, None]


def _tc_pre(x, W1, degp):
    return pl.pallas_call(
        _pre_body,
        grid=(GRID,),
        in_specs=[
            pl.BlockSpec((BLK, D), lambda i: (i, 0)),
            pl.BlockSpec((D, D), lambda i: (0, 0)),
            pl.BlockSpec((NC, NB, D), lambda i: (0, 0, 0)),
        ],
        out_specs=pl.BlockSpec((BLK, D), lambda i: (i, 0)),
        out_shape=jax.ShapeDtypeStruct((NP, D), jnp.float32),
    )(x, W1, degp)


def _mid_body(a_ref, degp_ref, b_ref, w_ref, h_ref):
    i = pl.program_id(0)
    dsl = pl.ds(i * BLK, BLK)
    no, ni = _norms(degp_ref, dsl)
    a = a_ref[0] + a_ref[1]
    t = jnp.maximum(a * ni[:, None] + b_ref[0], 0.0)
    y = jnp.dot(t, w_ref[...], preferred_element_type=jnp.float32)
    h_ref[...] = y * no[:, None]


def _tc_mid(agg1, degp, b1, W2):
    return pl.pallas_call(
        _mid_body,
        grid=(GRID,),
        in_specs=[
            pl.BlockSpec((NC, BLK, D), lambda i: (0, i, 0)),
            pl.BlockSpec((NC, NB, D), lambda i: (0, 0, 0)),
            pl.BlockSpec((1, D), lambda i: (0, 0)),
            pl.BlockSpec((D, D), lambda i: (0, 0)),
        ],
        out_specs=pl.BlockSpec((BLK, D), lambda i: (i, 0)),
        out_shape=jax.ShapeDtypeStruct((NP, D), jnp.float32),
    )(agg1, degp, b1.reshape(1, D), W2)


def _post_body(a_ref, degp_ref, b_ref, x_ref, o_ref):
    i = pl.program_id(0)
    dsl = pl.ds(i * BLK, BLK)
    _, ni = _norms(degp_ref, dsl)
    a = a_ref[0] + a_ref[1]
    o_ref[...] = jnp.maximum(a * ni[:, None] + b_ref[0], 0.0) + x_ref[...]


def _tc_post(agg2, degp, b2, x):
    return pl.pallas_call(
        _post_body,
        grid=(GRID,),
        in_specs=[
            pl.BlockSpec((NC, BLK, D), lambda i: (0, i, 0)),
            pl.BlockSpec((NC, NB, D), lambda i: (0, 0, 0)),
            pl.BlockSpec((1, D), lambda i: (0, 0)),
            pl.BlockSpec((BLK, D), lambda i: (i, 0)),
        ],
        out_specs=pl.BlockSpec((BLK, D), lambda i: (i, 0)),
        out_shape=jax.ShapeDtypeStruct((N, D), jnp.float32),
    )(agg2, degp, b2.reshape(1, D), x)


def kernel(x, edge_index, W1, b1, W2, b2):
    ei = edge_index.astype(jnp.int32)
    src, dst = ei[0], ei[1]
    degp = _deg_pass(src, dst)
    h1 = _tc_pre(x, W1, degp)
    agg1 = _edge_pass(h1, src, dst)
    h2 = _tc_mid(agg1, degp, b1, W2)
    agg2 = _edge_pass(h2, src, dst)
    return _tc_post(agg2, degp, b2, x)


# speedup vs baseline: 1.0135x; 1.0135x over previous
import jax
import jax.numpy as jnp
from jax import lax
from jax.experimental import pallas as pl
from jax.experimental.pallas import tpu as pltpu
from jax.experimental.pallas import tpu_sc as plsc

N = 10000
E = 320000
D = 128
NC = 2
NS = 16
NP = 10240
NB = NP
L = 16
K = 80

BLK = 2048
GRID = NP // BLK


def _mesh():
    return plsc.VectorSubcoreMesh(
        core_axis_name="c", subcore_axis_name="s", num_cores=NC, num_subcores=NS
    )


GB = 5
NG = (E // (NC * NS)) // K // GB


def _deg_body(src_hbm, dst_hbm, out_hbm, ibS, ibD, obS, obD, zb, isem, ssem, hist):
    c = lax.axis_index("c")
    s = lax.axis_index("s")
    oh0 = jnp.where(lax.iota(jnp.int32, L) == 0, 1.0, 0.0).astype(jnp.float32)
    oh1 = jnp.where(lax.iota(jnp.int32, L) == 1, 1.0, 0.0).astype(jnp.float32)
    zv = jnp.zeros((L,), jnp.float32)

    def fill_o(i, _):
        for j in range(D // L):
            obS[i, pl.ds(j * L, L)] = oh0 if j == 0 else zv
            obD[i, pl.ds(j * L, L)] = oh1 if j == 0 else zv
        return 0

    lax.fori_loop(0, K, fill_o, 0)

    def fill_z(i, _):
        for j in range(D // L):
            zb[i, pl.ds(j * L, L)] = zv
        return 0

    lax.fori_loop(0, 64, fill_z, 0)

    def fill_deg(i, _):
        pltpu.sync_copy(zb, hist.at[pl.ds(s * 640 + i * 64, 64), :])
        return 0

    lax.fori_loop(0, 10, fill_deg, 0)
    plsc.subcore_barrier()

    ept = E // (NC * NS)
    base_t = c * (E // NC) + s * ept

    def group(g, _):
        par = lax.rem(g, 2) * GB
        descs = []
        for b in range(GB):
            base = base_t + (g * GB + b) * K
            descs.append(pltpu.async_copy(src_hbm.at[pl.ds(base, K)], ibS.at[par + b], isem))
            descs.append(pltpu.async_copy(dst_hbm.at[pl.ds(base, K)], ibD.at[par + b], isem))

        for d_ in descs:
            d_.wait()
        for b in range(GB):
            pltpu.async_copy(obS, hist.at[ibS.at[par + b]], ssem, add=True)
            pltpu.async_copy(obD, hist.at[ibD.at[par + b]], ssem, add=True)

        @pl.when(g > 0)
        def _():
            for b in range(2 * GB):
                pltpu.make_async_copy(out_hbm.at[c, pl.ds(0, K), :], obS, ssem).wait()
        return 0

    lax.fori_loop(0, NG, group, 0)
    for b in range(2 * GB):
        pltpu.make_async_copy(out_hbm.at[c, pl.ds(0, K), :], obS, ssem).wait()
    plsc.subcore_barrier()
    sl = pl.ds(s * 640, 640)
    pltpu.sync_copy(hist.at[sl, :], out_hbm.at[c, sl, :])


def _deg_pass(src, dst):
    f = pl.kernel(
        _deg_body,
        out_type=jax.ShapeDtypeStruct((NC, NB, D), jnp.float32),
        mesh=_mesh(),
        scratch_types=[
            pltpu.VMEM((2 * GB, K), jnp.int32),
            pltpu.VMEM((2 * GB, K), jnp.int32),
            pltpu.VMEM((K, D), jnp.float32),
            pltpu.VMEM((K, D), jnp.float32),
            pltpu.VMEM((64, D), jnp.float32),
            pltpu.SemaphoreType.DMA,
            pltpu.SemaphoreType.DMA,
            pltpu.VMEM_SHARED((NB, D), jnp.float32),
        ],
    )
    return f(src, dst)


GBE = 4
NGE = 31


def _edge_body(h_hbm, src_hbm, dst_hbm, agg_hbm, sidx, didx, r0, r1, r2, r3,
               isem, gsem, ssem, asp):
    c = lax.axis_index("c")
    s = lax.axis_index("s")
    rows = [r0, r1, r2, r3]
    rpt = NP // NS
    zv = jnp.zeros((L,), jnp.float32)

    def fill_z(i, _):
        for j in range(D // L):
            r0[i, pl.ds(j * L, L)] = zv
        return 0

    lax.fori_loop(0, K, fill_z, 0)

    def fill_a(i, _):
        pltpu.sync_copy(r0, asp.at[pl.ds(s * rpt + i * K, K), :])
        return 0

    lax.fori_loop(0, 8, fill_a, 0)
    plsc.subcore_barrier()

    ept = E // (NC * NS)
    base_t = c * (E // NC) + s * ept

    def group(g, _):
        par = lax.rem(g, 2) * GBE
        descs = []
        for b in range(GBE):
            base = base_t + (g * GBE + b) * K
            descs.append(pltpu.async_copy(src_hbm.at[pl.ds(base, K)], sidx.at[par + b], isem))
            descs.append(pltpu.async_copy(dst_hbm.at[pl.ds(base, K)], didx.at[par + b], isem))
        for d_ in descs:
            d_.wait()
        gds = []
        for b in range(GBE):
            @pl.when(g > 0)
            def _():
                pltpu.make_async_copy(h_hbm.at[pl.ds(0, K), :], rows[b], ssem).wait()
            gds.append(pltpu.async_copy(h_hbm.at[sidx.at[par + b]], rows[b], gsem))
        for b in range(GBE):
            gds[b].wait()
            pltpu.async_copy(rows[b], asp.at[didx.at[par + b]], ssem, add=True)
        return 0

    lax.fori_loop(0, NGE, group, 0)
    for b in range(GBE):
        pltpu.make_async_copy(h_hbm.at[pl.ds(0, K), :], rows[b], ssem).wait()
    tbase = base_t + NGE * GBE * K
    pltpu.sync_copy(src_hbm.at[pl.ds(tbase, K)], sidx.at[0])
    pltpu.sync_copy(dst_hbm.at[pl.ds(tbase, K)], didx.at[0])
    pltpu.async_copy(h_hbm.at[sidx.at[0]], r0, gsem).wait()
    pltpu.async_copy(r0, asp.at[didx.at[0]], ssem, add=True).wait()
    plsc.subcore_barrier()
    sl = pl.ds(s * rpt, rpt)
    pltpu.sync_copy(asp.at[sl, :], agg_hbm.at[c, sl, :])


def _edge_pass(h, src, dst):
    f = pl.kernel(
        _edge_body,
        out_type=jax.ShapeDtypeStruct((NC, NP, D), jnp.float32),
        mesh=_mesh(),
        scratch_types=[
            pltpu.VMEM((2 * GBE, K), jnp.int32),
            pltpu.VMEM((2 * GBE, K), jnp.int32),
            pltpu.VMEM((K, D), jnp.float32),
            pltpu.VMEM((K, D), jnp.float32),
            pltpu.VMEM((K, D), jnp.float32),
            pltpu.VMEM((K, D), jnp.float32),
            pltpu.SemaphoreType.DMA,
            pltpu.SemaphoreType.DMA,
            pltpu.SemaphoreType.DMA,
            pltpu.VMEM_SHARED((NP, D), jnp.float32),
        ],
    )
    return f(h, src, dst)


def _norms(degp_ref, dsl):
    blk = degp_ref[0, dsl, :] + degp_ref[1, dsl, :]
    lanes = lax.broadcasted_iota(jnp.int32, (1, D), 1)
    d_out = jnp.sum(jnp.where(lanes == 0, blk, 0.0), axis=1)
    d_in = jnp.sum(jnp.where(lanes == 1, blk, 0.0), axis=1)
    no = jnp.where(d_out > 0, lax.rsqrt(jnp.maximum(d_out, 1.0)), 0.0)
    ni = jnp.where(d_in > 0, lax.rsqrt(jnp.maximum(d_in, 1.0)), 0.0)
    return no, ni


def _pre_body(x_ref, w_ref, degp_ref, h_ref, a_ref):
    i = pl.program_id(0)
    dsl = pl.ds(i * BLK, BLK)
    no, ni = _norms(degp_ref, dsl)
    y = jnp.dot(x_ref[...], w_ref[...], preferred_element_type=jnp.float32)
    h_ref[...] = y * no[:, None]
    a_ref[...] = jnp.concatenate(
        [no[:, None], ni[:, None], jnp.zeros((BLK, 6), jnp.float32)], axis=1
    )


def _tc_pre(x, W1, degp):
    return pl.pallas_call(
        _pre_body,
        grid=(GRID,),
        in_specs=[
            pl.BlockSpec((BLK, D), lambda i: (i, 0)),
            pl.BlockSpec((D, D), lambda i: (0, 0)),
            pl.BlockSpec((NC, NB, D), lambda i: (0, 0, 0)),
        ],
        out_specs=[
            pl.BlockSpec((BLK, D), lambda i: (i, 0)),
            pl.BlockSpec((BLK, 8), lambda i: (i, 0)),
        ],
        out_shape=[
            jax.ShapeDtypeStruct((NP, D), jnp.float32),
            jax.ShapeDtypeStruct((NP, 8), jnp.float32),
        ],
    )(x, W1, degp)


def _mid_body(a_ref, nrm_ref, b_ref, w_ref, h_ref):
    ni = nrm_ref[:, 1]
    no = nrm_ref[:, 0]
    a = a_ref[0] + a_ref[1]
    t = jnp.maximum(a * ni[:, None] + b_ref[0], 0.0)
    y = jnp.dot(t, w_ref[...], preferred_element_type=jnp.float32)
    h_ref[...] = y * no[:, None]


def _tc_mid(agg1, nrm, b1, W2):
    return pl.pallas_call(
        _mid_body,
        grid=(GRID,),
        in_specs=[
            pl.BlockSpec((NC, BLK, D), lambda i: (0, i, 0)),
            pl.BlockSpec((BLK, 8), lambda i: (i, 0)),
            pl.BlockSpec((1, D), lambda i: (0, 0)),
            pl.BlockSpec((D, D), lambda i: (0, 0)),
        ],
        out_specs=pl.BlockSpec((BLK, D), lambda i: (i, 0)),
        out_shape=jax.ShapeDtypeStruct((NP, D), jnp.float32),
    )(agg1, nrm, b1.reshape(1, D), W2)


def _post_body(a_ref, nrm_ref, b_ref, x_ref, o_ref):
    ni = nrm_ref[:, 1]
    a = a_ref[0] + a_ref[1]
    o_ref[...] = jnp.maximum(a * ni[:, None] + b_ref[0], 0.0) + x_ref[...]


def _tc_post(agg2, nrm, b2, x):
    return pl.pallas_call(
        _post_body,
        grid=(GRID,),
        in_specs=[
            pl.BlockSpec((NC, BLK, D), lambda i: (0, i, 0)),
            pl.BlockSpec((BLK, 8), lambda i: (i, 0)),
            pl.BlockSpec((1, D), lambda i: (0, 0)),
            pl.BlockSpec((BLK, D), lambda i: (i, 0)),
        ],
        out_specs=pl.BlockSpec((BLK, D), lambda i: (i, 0)),
        out_shape=jax.ShapeDtypeStruct((N, D), jnp.float32),
    )(agg2, nrm, b2.reshape(1, D), x)


def kernel(x, edge_index, W1, b1, W2, b2):
    ei = edge_index.astype(jnp.int32)
    src, dst = ei[0], ei[1]
    degp = _deg_pass(src, dst)
    h1, nrm = _tc_pre(x, W1, degp)
    agg1 = _edge_pass(h1, src, dst)
    h2 = _tc_mid(agg1, nrm, b1, W2)
    agg2 = _edge_pass(h2, src, dst)
    return _tc_post(agg2, nrm, b2, x)
